# baseline (speedup 1.0000x reference)
import functools

import jax
import jax.numpy as jnp
from jax import lax
from jax.experimental import pallas as pl
from jax.experimental.pallas import tpu as pltpu
from jax.experimental.pallas import tpu_sc as plsc

D_MODEL = 64
LANES = 16
NUM_WORKERS = 32
IDX_MINOR = 128
K_ROWS = 8


@functools.partial(jax.jit, static_argnames=())
def _sc_gather(ids2d, table):
    R = ids2d.shape[0]
    rows_per_w = R // NUM_WORKERS
    steps = rows_per_w // K_ROWS
    chunk_ids = K_ROWS * IDX_MINOR

    mesh = plsc.VectorSubcoreMesh(core_axis_name="c", subcore_axis_name="s")

    @functools.partial(
        pl.kernel,
        mesh=mesh,
        compiler_params=pltpu.CompilerParams(use_tc_tiling_on_sc=False),
        out_type=jax.ShapeDtypeStruct((R * IDX_MINOR, D_MODEL), jnp.float32),
        scratch_types=[
            pltpu.VMEM((K_ROWS, IDX_MINOR), jnp.int32),
            pltpu.VMEM((chunk_ids, D_MODEL), jnp.float32),
            pltpu.SemaphoreType.DMA,
        ],
    )
    def k(ids_hbm, tab_hbm, out_hbm, idx_v, rows_v, sem):
        cid = lax.axis_index("c")
        sid = lax.axis_index("s")
        wid = sid * 2 + cid
        rbase = wid * rows_per_w

        def step(i, carry):
            r0 = rbase + i * K_ROWS
            pltpu.sync_copy(ids_hbm.at[pl.ds(r0, K_ROWS)], idx_v)
            for j in range(K_ROWS):
                for t in range(IDX_MINOR // LANES):
                    sl = (j, pl.ds(t * LANES, LANES))
                    idx_v[sl] = idx_v[sl] + 1
            for j in range(K_ROWS):
                pltpu.async_copy(
                    tab_hbm.at[idx_v.at[j]],
                    rows_v.at[pl.ds(j * IDX_MINOR, IDX_MINOR)],
                    sem,
                ).wait()
            pltpu.sync_copy(rows_v, out_hbm.at[pl.ds(r0 * IDX_MINOR, chunk_ids)])
            return carry

        lax.fori_loop(0, steps, step, 0)

    return k(ids2d, table)


def kernel(dense_ids, emb_weight):
    batch, hist = dense_ids.shape
    ids2d = dense_ids.astype(jnp.int32).reshape(-1, IDX_MINOR)
    out = _sc_gather(ids2d, emb_weight)
    return out.reshape(batch, hist, D_MODEL)


# speedup vs baseline: 1.1176x; 1.1176x over previous
import functools

import jax
import jax.numpy as jnp
from jax import lax
from jax.experimental import pallas as pl
from jax.experimental.pallas import tpu as pltpu
from jax.experimental.pallas import tpu_sc as plsc

D_MODEL = 64
LANES = 16
NUM_WORKERS = 32
IDX_MINOR = 128
CHUNK_ROWS = 4
CHUNK_IDS = CHUNK_ROWS * IDX_MINOR
NBUF = 2


def _sc_gather(ids2d, table):
    R = ids2d.shape[0]
    rows_per_w = R // NUM_WORKERS
    steps = rows_per_w // CHUNK_ROWS
    pairs = steps // NBUF

    mesh = plsc.VectorSubcoreMesh(core_axis_name="c", subcore_axis_name="s")

    @functools.partial(
        pl.kernel,
        mesh=mesh,
        compiler_params=pltpu.CompilerParams(use_tc_tiling_on_sc=False),
        out_type=jax.ShapeDtypeStruct((R * IDX_MINOR, D_MODEL), jnp.float32),
        scratch_types=[
            pltpu.VMEM((rows_per_w, IDX_MINOR), jnp.int32),
            pltpu.VMEM((NBUF, CHUNK_IDS, D_MODEL), jnp.float32),
            pltpu.SemaphoreType.DMA((NBUF,)),
            pltpu.SemaphoreType.DMA((NBUF,)),
        ],
    )
    def k(ids_hbm, tab_hbm, out_hbm, idx_v, rows_v, gat_sem, out_sem):
        cid = lax.axis_index("c")
        sid = lax.axis_index("s")
        wid = sid * 2 + cid
        rbase = wid * rows_per_w

        pltpu.sync_copy(ids_hbm.at[pl.ds(rbase, rows_per_w)], idx_v)

        def shift_ids(i):
            for j in range(CHUNK_ROWS):
                for t in range(IDX_MINOR // LANES):
                    sl = (i * CHUNK_ROWS + j, pl.ds(t * LANES, LANES))
                    idx_v[sl] = idx_v[sl] + 1

        def fire_gathers(i, b):
            for j in range(CHUNK_ROWS):
                pltpu.async_copy(
                    tab_hbm.at[idx_v.at[i * CHUNK_ROWS + j]],
                    rows_v.at[b, pl.ds(j * IDX_MINOR, IDX_MINOR)],
                    gat_sem.at[b],
                )

        def drain_gathers(b):
            for j in range(CHUNK_ROWS):
                pltpu.make_async_copy(
                    tab_hbm.at[idx_v.at[j]],
                    rows_v.at[b, pl.ds(j * IDX_MINOR, IDX_MINOR)],
                    gat_sem.at[b],
                ).wait()

        def out_start(i, b):
            r0 = rbase + i * CHUNK_ROWS
            pltpu.async_copy(
                rows_v.at[b], out_hbm.at[pl.ds(r0 * IDX_MINOR, CHUNK_IDS)],
                out_sem.at[b])

        def out_wait(b):
            pltpu.make_async_copy(
                rows_v.at[b], out_hbm.at[pl.ds(0, CHUNK_IDS)], out_sem.at[b]
            ).wait()

        def pair_body(p, carry):
            for b in range(NBUF):
                i = p * NBUF + b
                shift_ids(i)
                @pl.when(p >= 1)
                def _():
                    out_wait(b)
                fire_gathers(i, b)
                if b == 0:
                    @pl.when(p >= 1)
                    def _():
                        drain_gathers(1)
                        out_start(i - 1, 1)
                else:
                    drain_gathers(0)
                    out_start(i - 1, 0)
            return carry

        lax.fori_loop(0, pairs, pair_body, 0)

        drain_gathers(1)
        out_start(steps - 1, 1)
        out_wait(0)
        out_wait(1)

    return k(ids2d, table)


def kernel(dense_ids, emb_weight):
    batch, hist = dense_ids.shape
    ids2d = dense_ids.astype(jnp.int32).reshape(-1, IDX_MINOR)
    out = _sc_gather(ids2d, emb_weight)
    return out.reshape(batch, hist, D_MODEL)


# speedup vs baseline: 1.3642x; 1.2207x over previous
import functools

import jax
import jax.numpy as jnp
from jax import lax
from jax.experimental import pallas as pl
from jax.experimental.pallas import tpu as pltpu
from jax.experimental.pallas import tpu_sc as plsc

D_MODEL = 64
LANES = 16
NUM_WORKERS = 32
IDX_MINOR = 128
PAD_W = 128
CHUNK_ROWS = 2
CHUNK_IDS = CHUNK_ROWS * IDX_MINOR
NBUF = 2


def _sc_gather(ids2d, tab_pad):
    R = ids2d.shape[0]
    rows_per_w = R // NUM_WORKERS
    steps = rows_per_w // CHUNK_ROWS
    pairs = steps // NBUF

    mesh = plsc.VectorSubcoreMesh(core_axis_name="c", subcore_axis_name="s")

    @functools.partial(
        pl.kernel,
        mesh=mesh,
        out_type=jax.ShapeDtypeStruct((R * IDX_MINOR, PAD_W), jnp.float32),
        scratch_types=[
            pltpu.VMEM((rows_per_w, IDX_MINOR), jnp.int32),
            pltpu.VMEM((NBUF, CHUNK_IDS, PAD_W), jnp.float32),
            pltpu.SemaphoreType.DMA((NBUF,)),
            pltpu.SemaphoreType.DMA((NBUF,)),
        ],
    )
    def k(ids_hbm, tab_hbm, out_hbm, idx_v, rows_v, gat_sem, out_sem):
        cid = lax.axis_index("c")
        sid = lax.axis_index("s")
        wid = sid * 2 + cid
        rbase = wid * rows_per_w

        pltpu.sync_copy(ids_hbm.at[pl.ds(rbase, rows_per_w)], idx_v)

        def shift_ids(i):
            for j in range(CHUNK_ROWS):
                for t in range(IDX_MINOR // LANES):
                    sl = (i * CHUNK_ROWS + j, pl.ds(t * LANES, LANES))
                    idx_v[sl] = idx_v[sl] + 1

        def fire_gathers(i, b):
            for j in range(CHUNK_ROWS):
                pltpu.async_copy(
                    tab_hbm.at[idx_v.at[i * CHUNK_ROWS + j]],
                    rows_v.at[b, pl.ds(j * IDX_MINOR, IDX_MINOR)],
                    gat_sem.at[b],
                )

        def drain_gathers(b):
            for j in range(CHUNK_ROWS):
                pltpu.make_async_copy(
                    tab_hbm.at[idx_v.at[j]],
                    rows_v.at[b, pl.ds(j * IDX_MINOR, IDX_MINOR)],
                    gat_sem.at[b],
                ).wait()

        def out_start(i, b):
            r0 = rbase + i * CHUNK_ROWS
            pltpu.async_copy(
                rows_v.at[b],
                out_hbm.at[pl.ds(r0 * IDX_MINOR, CHUNK_IDS)],
                out_sem.at[b])

        def out_wait(b):
            pltpu.make_async_copy(
                rows_v.at[b],
                out_hbm.at[pl.ds(0, CHUNK_IDS)],
                out_sem.at[b],
            ).wait()

        def pair_body(p, carry):
            for b in range(NBUF):
                i = p * NBUF + b
                shift_ids(i)
                @pl.when(p >= 1)
                def _():
                    out_wait(b)
                fire_gathers(i, b)
                if b == 0:
                    @pl.when(p >= 1)
                    def _():
                        drain_gathers(1)
                        out_start(i - 1, 1)
                else:
                    drain_gathers(0)
                    out_start(i - 1, 0)
            return carry

        lax.fori_loop(0, pairs, pair_body, 0)

        drain_gathers(1)
        out_start(steps - 1, 1)
        out_wait(0)
        out_wait(1)

    return k(ids2d, tab_pad)


def kernel(dense_ids, emb_weight):
    batch, hist = dense_ids.shape
    ids2d = dense_ids.astype(jnp.int32).reshape(-1, IDX_MINOR)
    n_rows = emb_weight.shape[0]
    pad_rows = (-n_rows) % 8
    tab_pad = jnp.pad(
        emb_weight, ((0, pad_rows), (0, PAD_W - emb_weight.shape[1])))
    out = _sc_gather(ids2d, tab_pad)
    return out[:, :D_MODEL].reshape(batch, hist, D_MODEL)


# speedup vs baseline: 1.3652x; 1.0008x over previous
import functools

import jax
import jax.numpy as jnp
from jax import lax
from jax.experimental import pallas as pl
from jax.experimental.pallas import tpu as pltpu
from jax.experimental.pallas import tpu_sc as plsc

D_MODEL = 64
LANES = 16
NUM_WORKERS = 32
IDX_MINOR = 128
PAD_W = 128
CHUNK_ROWS = 2
CHUNK_IDS = CHUNK_ROWS * IDX_MINOR
NBUF = 2


def _sc_gather(ids2d, tab_pad):
    R = ids2d.shape[0]
    rows_per_w = R // NUM_WORKERS
    steps = rows_per_w // CHUNK_ROWS
    pairs = steps // NBUF

    mesh = plsc.VectorSubcoreMesh(core_axis_name="c", subcore_axis_name="s")

    @functools.partial(
        pl.kernel,
        mesh=mesh,
        out_type=jax.ShapeDtypeStruct((R * IDX_MINOR, PAD_W), jnp.float32),
        scratch_types=[
            pltpu.VMEM((rows_per_w, IDX_MINOR), jnp.int32),
            pltpu.VMEM((NBUF, CHUNK_IDS, PAD_W), jnp.float32),
            pltpu.SemaphoreType.DMA((NBUF,)),
            pltpu.SemaphoreType.DMA((NBUF,)),
        ],
    )
    def k(ids_hbm, tab_hbm, out_hbm, idx_v, rows_v, gat_sem, out_sem):
        cid = lax.axis_index("c")
        sid = lax.axis_index("s")
        wid = sid * 2 + cid
        rbase = wid * rows_per_w

        pltpu.sync_copy(ids_hbm.at[pl.ds(rbase, rows_per_w)], idx_v)

        def shift_ids(i):
            for j in range(CHUNK_ROWS):
                for t in range(IDX_MINOR // LANES):
                    sl = (i * CHUNK_ROWS + j, pl.ds(t * LANES, LANES))
                    idx_v[sl] = idx_v[sl] + 1

        def fire_gathers(i, b):
            for j in range(CHUNK_ROWS):
                pltpu.async_copy(
                    tab_hbm.at[idx_v.at[i * CHUNK_ROWS + j]],
                    rows_v.at[b, pl.ds(j * IDX_MINOR, IDX_MINOR)],
                    gat_sem.at[b],
                )

        def drain_gathers(b):
            for j in range(CHUNK_ROWS):
                pltpu.make_async_copy(
                    tab_hbm.at[idx_v.at[j]],
                    rows_v.at[b, pl.ds(j * IDX_MINOR, IDX_MINOR)],
                    gat_sem.at[b],
                ).wait()

        def drain_and_out(i, b):
            r0 = rbase + i * CHUNK_ROWS
            for j in range(CHUNK_ROWS):
                pltpu.make_async_copy(
                    tab_hbm.at[idx_v.at[j]],
                    rows_v.at[b, pl.ds(j * IDX_MINOR, IDX_MINOR)],
                    gat_sem.at[b],
                ).wait()
                pltpu.async_copy(
                    rows_v.at[b, pl.ds(j * IDX_MINOR, IDX_MINOR)],
                    out_hbm.at[pl.ds((r0 + j) * IDX_MINOR, IDX_MINOR)],
                    out_sem.at[b])

        def out_start(i, b):
            r0 = rbase + i * CHUNK_ROWS
            pltpu.async_copy(
                rows_v.at[b],
                out_hbm.at[pl.ds(r0 * IDX_MINOR, CHUNK_IDS)],
                out_sem.at[b])

        def out_wait(b):
            pltpu.make_async_copy(
                rows_v.at[b],
                out_hbm.at[pl.ds(0, CHUNK_IDS)],
                out_sem.at[b],
            ).wait()

        def pair_body(p, carry):
            for b in range(NBUF):
                i = p * NBUF + b
                shift_ids(i)
                @pl.when(p >= 1)
                def _():
                    out_wait(b)
                fire_gathers(i, b)
                if b == 0:
                    @pl.when(p >= 1)
                    def _():
                        drain_and_out(i - 1, 1)
                else:
                    drain_and_out(i - 1, 0)
            return carry

        lax.fori_loop(0, pairs, pair_body, 0)

        drain_and_out(steps - 1, 1)
        out_wait(0)
        out_wait(1)

    return k(ids2d, tab_pad)


def kernel(dense_ids, emb_weight):
    batch, hist = dense_ids.shape
    ids2d = dense_ids.astype(jnp.int32).reshape(-1, IDX_MINOR)
    n_rows = emb_weight.shape[0]
    pad_rows = (-n_rows) % 8
    tab_pad = jnp.pad(
        emb_weight, ((0, pad_rows), (0, PAD_W - emb_weight.shape[1])))
    out = _sc_gather(ids2d, tab_pad)
    return out[:, :D_MODEL].reshape(batch, hist, D_MODEL)
